# baseline (speedup 1.0000x reference)
import functools

import jax
import jax.numpy as jnp
from jax import lax
from jax.experimental import pallas as pl
from jax.experimental.pallas import tpu as pltpu
from jax.experimental.pallas import tpu_sc as plsc


_T = 8192
_D = 1024
_INFO = plsc.get_sparse_core_info()
_NW = _INFO.num_cores * _INFO.num_subcores
_ROWS_PER_W = _T // _NW
_CHUNK = 32
_NBUF = 3
_NCHUNKS = _ROWS_PER_W // _CHUNK


@functools.partial(
    pl.kernel,
    mesh=plsc.VectorSubcoreMesh(core_axis_name="c", subcore_axis_name="s"),
    out_type=jax.ShapeDtypeStruct((1, _T, _D), jnp.float32),
    scratch_types=[
        pltpu.VMEM((_NBUF, _CHUNK, _D), jnp.float32),
        pltpu.SemaphoreType.DMA((_NBUF,)),
        pltpu.SemaphoreType.DMA((_NBUF,)),
    ],
)
def _sc_copy(emb_hbm, out_hbm, buf, in_sems, out_sems):
    wid = lax.axis_index("s") * _INFO.num_cores + lax.axis_index("c")
    base = wid * _ROWS_PER_W

    def in_copy(i, slot):
        return pltpu.make_async_copy(
            emb_hbm.at[pl.ds(base + i * _CHUNK, _CHUNK), :],
            buf.at[slot],
            in_sems.at[slot],
        )

    def out_copy(i, slot):
        return pltpu.make_async_copy(
            buf.at[slot],
            out_hbm.at[0, pl.ds(base + i * _CHUNK, _CHUNK), :],
            out_sems.at[slot],
        )

    in_copy(0, 0).start()
    for i in range(_NCHUNKS):
        slot = i % _NBUF
        if i + 1 < _NCHUNKS:
            nslot = (i + 1) % _NBUF
            if i + 1 >= _NBUF:
                out_copy(i + 1 - _NBUF, nslot).wait()
            in_copy(i + 1, nslot).start()
        in_copy(i, slot).wait()
        out_copy(i, slot).start()
    for i in range(max(0, _NCHUNKS - _NBUF), _NCHUNKS):
        out_copy(i, i % _NBUF).wait()


def kernel(x, emb):
    del x
    return _sc_copy(emb)


# speedup vs baseline: 1.0088x; 1.0088x over previous
import functools

import jax
import jax.numpy as jnp
from jax import lax
from jax.experimental import pallas as pl
from jax.experimental.pallas import tpu as pltpu
from jax.experimental.pallas import tpu_sc as plsc


_T = 8192
_D = 1024
_NC = 2
_ROWS_PER_C = _T // _NC
_CHUNK = 512
_NBUF = 3
_NCHUNKS = _ROWS_PER_C // _CHUNK


@functools.partial(
    pl.kernel,
    mesh=plsc.ScalarSubcoreMesh(axis_name="c", num_cores=_NC),
    out_type=jax.ShapeDtypeStruct((1, _T, _D), jnp.float32),
    scratch_types=[
        pltpu.VMEM_SHARED((_NBUF, _CHUNK, _D), jnp.float32),
        pltpu.SemaphoreType.DMA((_NBUF,)),
        pltpu.SemaphoreType.DMA((_NBUF,)),
    ],
)
def _sc_copy(emb_hbm, out_hbm, buf, in_sems, out_sems):
    base = lax.axis_index("c") * _ROWS_PER_C

    def in_copy(i, slot):
        return pltpu.make_async_copy(
            emb_hbm.at[pl.ds(base + i * _CHUNK, _CHUNK), :],
            buf.at[slot],
            in_sems.at[slot],
        )

    def out_copy(i, slot):
        return pltpu.make_async_copy(
            buf.at[slot],
            out_hbm.at[0, pl.ds(base + i * _CHUNK, _CHUNK), :],
            out_sems.at[slot],
        )

    in_copy(0, 0).start()
    for i in range(_NCHUNKS):
        slot = i % _NBUF
        if i + 1 < _NCHUNKS:
            nslot = (i + 1) % _NBUF
            if i + 1 >= _NBUF:
                out_copy(i + 1 - _NBUF, nslot).wait()
            in_copy(i + 1, nslot).start()
        in_copy(i, slot).wait()
        out_copy(i, slot).start()
    for i in range(max(0, _NCHUNKS - _NBUF), _NCHUNKS):
        out_copy(i, i % _NBUF).wait()


def kernel(x, emb):
    del x
    return _sc_copy(emb)


# speedup vs baseline: 2.0351x; 2.0173x over previous
import jax
import jax.numpy as jnp
from jax.experimental import pallas as pl
from jax.experimental.pallas import tpu as pltpu


_CHUNK = 512
_NBUF = 8


def _relay_body(emb_ref, out_ref, buf, in_sems, out_sems):
    rows = emb_ref.shape[0]
    nchunks = rows // _CHUNK

    def in_copy(i, slot):
        return pltpu.make_async_copy(
            emb_ref.at[pl.ds(i * _CHUNK, _CHUNK), :],
            buf.at[slot],
            in_sems.at[slot],
        )

    def out_copy(i, slot):
        return pltpu.make_async_copy(
            buf.at[slot],
            out_ref.at[0, pl.ds(i * _CHUNK, _CHUNK), :],
            out_sems.at[slot],
        )

    lag = _NBUF // 2
    for i in range(min(_NBUF, nchunks)):
        in_copy(i, i).start()
    for i in range(nchunks + lag):
        if i < nchunks:
            slot = i % _NBUF
            in_copy(i, slot).wait()
            out_copy(i, slot).start()
        j = i - lag
        if 0 <= j and j + _NBUF < nchunks:
            out_copy(j, j % _NBUF).wait()
            in_copy(j + _NBUF, j % _NBUF).start()
    for i in range(max(0, nchunks - _NBUF), nchunks):
        out_copy(i, i % _NBUF).wait()


def kernel(x, emb):
    del x
    T, D = emb.shape
    out = pl.pallas_call(
        _relay_body,
        in_specs=[pl.BlockSpec(memory_space=pltpu.MemorySpace.HBM)],
        out_specs=pl.BlockSpec(memory_space=pltpu.MemorySpace.HBM),
        out_shape=jax.ShapeDtypeStruct((1, T, D), emb.dtype),
        scratch_shapes=[
            pltpu.VMEM((_NBUF, _CHUNK, D), jnp.float32),
            pltpu.SemaphoreType.DMA((_NBUF,)),
            pltpu.SemaphoreType.DMA((_NBUF,)),
        ],
    )(emb)
    return out
